# Initial kernel scaffold; baseline (speedup 1.0000x reference)
import jax
import jax.numpy as jnp
from jax.experimental import pallas as pl


def kernel(x, embed):
    raise NotImplementedError("write your pallas kernel here")


# speedup vs baseline: 2.1253x; 2.1253x over previous
import functools

import jax
import jax.numpy as jnp
from jax import lax
from jax.experimental import pallas as pl
from jax.experimental.pallas import tpu as pltpu
from jax.experimental.pallas import tpu_sc as plsc

_BLK = 512


def _dist_argmax_body(x_ref, e_ref, dist_ref, ind_ref):
    xb = x_ref[...]
    eb = e_ref[...]
    x2 = jnp.sum(xb * xb, axis=1, keepdims=True)
    e2 = jnp.sum(eb * eb, axis=1)
    xe = lax.dot_general(xb, eb, (((1,), (1,)), ((), ())),
                         preferred_element_type=jnp.float32)
    dist = -((x2 - 2.0 * xe) + e2)
    dist_ref[...] = dist
    k = dist.shape[1]
    m = jnp.max(dist, axis=1, keepdims=True)
    iota = lax.broadcasted_iota(jnp.int32, dist.shape, 1)
    ind = jnp.min(jnp.where(dist == m, iota, k), axis=1, keepdims=True)
    ind_ref[...] = ind


def _dist_argmax(x_flat, embed2d):
    bn, d = x_flat.shape
    k = embed2d.shape[0]
    grid = (bn // _BLK,)
    dist, ind = pl.pallas_call(
        _dist_argmax_body,
        grid=grid,
        in_specs=[
            pl.BlockSpec((_BLK, d), lambda i: (i, 0)),
            pl.BlockSpec((k, d), lambda i: (0, 0)),
        ],
        out_specs=[
            pl.BlockSpec((_BLK, k), lambda i: (i, 0)),
            pl.BlockSpec((_BLK, 1), lambda i: (i, 0)),
        ],
        out_shape=[
            jax.ShapeDtypeStruct((bn, k), jnp.float32),
            jax.ShapeDtypeStruct((bn, 1), jnp.int32),
        ],
    )(x_flat, embed2d)
    return dist, ind.reshape(bn)


def _sc_gather(table, idx):
    info = plsc.get_sparse_core_info()
    nc, ns = info.num_cores, info.num_subcores
    nw = nc * ns
    bn = idx.shape[0]
    d = table.shape[1]
    b_per_w = bn // nw
    mesh = plsc.VectorSubcoreMesh(core_axis_name="c", subcore_axis_name="s")

    @functools.partial(
        pl.kernel, mesh=mesh,
        out_type=jax.ShapeDtypeStruct((bn, d), jnp.float32),
        scratch_types=[
            pltpu.VMEM((b_per_w,), jnp.int32),
            pltpu.VMEM((b_per_w, d), jnp.float32),
            pltpu.SemaphoreType.DMA,
        ],
        compiler_params=pltpu.CompilerParams(use_tc_tiling_on_sc=False),
    )
    def gk(table_hbm, idx_hbm, out_hbm, idx_v, rows_v, sem):
        wid = lax.axis_index("s") * nc + lax.axis_index("c")
        base = wid * b_per_w
        pltpu.sync_copy(idx_hbm.at[pl.ds(base, b_per_w)], idx_v)
        pltpu.async_copy(table_hbm.at[idx_v], rows_v, sem).wait()
        pltpu.sync_copy(rows_v, out_hbm.at[pl.ds(base, b_per_w)])

    return gk(table, idx)


def kernel(x, embed):
    b, n, d = x.shape
    h, k, _ = embed.shape
    x_flat = x.reshape(b * n, d).astype(jnp.float32)
    embed2d = embed.reshape(k, d)
    dist2d, ind = _dist_argmax(x_flat, embed2d)
    quantize = _sc_gather(embed2d, ind)
    return (quantize.reshape(b, n, d),
            ind.reshape(b, n),
            dist2d.reshape(h, b * n, k))


# speedup vs baseline: 2.6236x; 1.2345x over previous
import functools

import jax
import jax.numpy as jnp
from jax import lax
from jax.experimental import pallas as pl
from jax.experimental.pallas import tpu as pltpu
from jax.experimental.pallas import tpu_sc as plsc

_BLK = 1024


def _dist_argmax_body(x_ref, e_ref, dist_ref, ind_ref):
    xb = x_ref[...]
    eb = e_ref[...]
    x2 = jnp.sum(xb * xb, axis=1, keepdims=True)
    e2 = jnp.sum(eb * eb, axis=1)
    xe = lax.dot_general(xb, eb, (((1,), (1,)), ((), ())),
                         preferred_element_type=jnp.float32)
    dist = -((x2 - 2.0 * xe) + e2)
    dist_ref[...] = dist
    k = dist.shape[1]
    m = jnp.max(dist, axis=1, keepdims=True)
    iota = lax.broadcasted_iota(jnp.int32, dist.shape, 1)
    ind = jnp.min(jnp.where(dist == m, iota, k), axis=1)
    ind_ref[...] = ind.reshape(ind_ref.shape)


def _dist_argmax(x_flat, embed2d):
    bn, d = x_flat.shape
    k = embed2d.shape[0]
    grid = (bn // _BLK,)
    dist, ind = pl.pallas_call(
        _dist_argmax_body,
        grid=grid,
        in_specs=[
            pl.BlockSpec((_BLK, d), lambda i: (i, 0)),
            pl.BlockSpec((k, d), lambda i: (0, 0)),
        ],
        out_specs=[
            pl.BlockSpec((_BLK, k), lambda i: (i, 0)),
            pl.BlockSpec((_BLK // 128, 128), lambda i: (i, 0)),
        ],
        out_shape=[
            jax.ShapeDtypeStruct((bn, k), jnp.float32),
            jax.ShapeDtypeStruct((bn // 128, 128), jnp.int32),
        ],
    )(x_flat, embed2d)
    return dist, ind.reshape(bn)


def _sc_gather(table, idx):
    info = plsc.get_sparse_core_info()
    nc, ns = info.num_cores, info.num_subcores
    nw = nc * ns
    bn = idx.shape[0]
    d = table.shape[1]
    b_per_w = bn // nw
    mesh = plsc.VectorSubcoreMesh(core_axis_name="c", subcore_axis_name="s")

    @functools.partial(
        pl.kernel, mesh=mesh,
        out_type=jax.ShapeDtypeStruct((bn, d), jnp.float32),
        scratch_types=[
            pltpu.VMEM((b_per_w,), jnp.int32),
            pltpu.VMEM((b_per_w, d), jnp.float32),
            pltpu.SemaphoreType.DMA,
        ],
        compiler_params=pltpu.CompilerParams(use_tc_tiling_on_sc=False),
    )
    def gk(table_hbm, idx_hbm, out_hbm, idx_v, rows_v, sem):
        wid = lax.axis_index("s") * nc + lax.axis_index("c")
        base = wid * b_per_w
        pltpu.sync_copy(idx_hbm.at[pl.ds(base, b_per_w)], idx_v)
        pltpu.async_copy(table_hbm.at[idx_v], rows_v, sem).wait()
        pltpu.sync_copy(rows_v, out_hbm.at[pl.ds(base, b_per_w)])

    return gk(table, idx)


def kernel(x, embed):
    b, n, d = x.shape
    h, k, _ = embed.shape
    x_flat = x.reshape(b * n, d).astype(jnp.float32)
    embed2d = embed.reshape(k, d)
    dist2d, ind = _dist_argmax(x_flat, embed2d)
    quantize = _sc_gather(embed2d, ind)
    return (quantize.reshape(b, n, d),
            ind.reshape(b, n),
            dist2d.reshape(h, b * n, k))


# speedup vs baseline: 2.7239x; 1.0382x over previous
import functools

import jax
import jax.numpy as jnp
from jax import lax
from jax.experimental import pallas as pl
from jax.experimental.pallas import tpu as pltpu
from jax.experimental.pallas import tpu_sc as plsc

_BLK = 1024


def _dist_argmax_body(x_ref, e_ref, dist_ref, ind_ref):
    xb = x_ref[...]
    eb = e_ref[...]
    x2 = jnp.sum(xb * xb, axis=1, keepdims=True)
    e2 = jnp.sum(eb * eb, axis=1)
    xe2 = lax.dot_general(xb + xb, eb, (((1,), (1,)), ((), ())),
                          preferred_element_type=jnp.float32)
    dist = (xe2 - x2) - e2
    dist_ref[...] = dist
    ind = jnp.argmax(dist, axis=1).astype(jnp.int32)
    ind_ref[...] = ind.reshape(ind_ref.shape)


def _dist_argmax(x_flat, embed2d):
    bn, d = x_flat.shape
    k = embed2d.shape[0]
    grid = (bn // _BLK,)
    dist, ind = pl.pallas_call(
        _dist_argmax_body,
        grid=grid,
        in_specs=[
            pl.BlockSpec((_BLK, d), lambda i: (i, 0)),
            pl.BlockSpec((k, d), lambda i: (0, 0)),
        ],
        out_specs=[
            pl.BlockSpec((_BLK, k), lambda i: (i, 0)),
            pl.BlockSpec((_BLK // 128, 128), lambda i: (i, 0)),
        ],
        out_shape=[
            jax.ShapeDtypeStruct((bn, k), jnp.float32),
            jax.ShapeDtypeStruct((bn // 128, 128), jnp.int32),
        ],
    )(x_flat, embed2d)
    return dist, ind.reshape(bn)


def _sc_gather(table, idx):
    info = plsc.get_sparse_core_info()
    nc, ns = info.num_cores, info.num_subcores
    nw = nc * ns
    bn = idx.shape[0]
    d = table.shape[1]
    b_per_w = bn // nw
    mesh = plsc.VectorSubcoreMesh(core_axis_name="c", subcore_axis_name="s")

    @functools.partial(
        pl.kernel, mesh=mesh,
        out_type=jax.ShapeDtypeStruct((bn, d), jnp.float32),
        scratch_types=[
            pltpu.VMEM((b_per_w,), jnp.int32),
            pltpu.VMEM((b_per_w, d), jnp.float32),
            pltpu.SemaphoreType.DMA,
        ],
        compiler_params=pltpu.CompilerParams(use_tc_tiling_on_sc=False),
    )
    def gk(table_hbm, idx_hbm, out_hbm, idx_v, rows_v, sem):
        wid = lax.axis_index("s") * nc + lax.axis_index("c")
        base = wid * b_per_w
        pltpu.sync_copy(idx_hbm.at[pl.ds(base, b_per_w)], idx_v)
        pltpu.async_copy(table_hbm.at[idx_v], rows_v, sem).wait()
        pltpu.sync_copy(rows_v, out_hbm.at[pl.ds(base, b_per_w)])

    return gk(table, idx)


def kernel(x, embed):
    b, n, d = x.shape
    h, k, _ = embed.shape
    x_flat = x.reshape(b * n, d).astype(jnp.float32)
    embed2d = embed.reshape(k, d)
    dist2d, ind = _dist_argmax(x_flat, embed2d)
    quantize = _sc_gather(embed2d, ind)
    return (quantize.reshape(b, n, d),
            ind.reshape(b, n),
            dist2d.reshape(h, b * n, k))


# speedup vs baseline: 2.7653x; 1.0152x over previous
import functools

import jax
import jax.numpy as jnp
from jax import lax
from jax.experimental import pallas as pl
from jax.experimental.pallas import tpu as pltpu
from jax.experimental.pallas import tpu_sc as plsc

_BLK = 1024


def _dist_argmax_body(x_ref, e_ref, dist_ref, ind_ref):
    xb = x_ref[...]
    eb = e_ref[...]
    x2 = jnp.sum(xb * xb, axis=1, keepdims=True)
    e2 = jnp.sum(eb * eb, axis=1)
    xe2 = lax.dot_general(xb + xb, eb, (((1,), (1,)), ((), ())),
                          preferred_element_type=jnp.float32)
    dist = (xe2 - x2) - e2
    dist_ref[...] = dist
    k = dist.shape[1]
    m = jnp.max(dist, axis=1, keepdims=True)
    iota = lax.broadcasted_iota(jnp.int32, dist.shape, 1).astype(jnp.float32)
    ind = jnp.min(jnp.where(dist == m, iota, float(k)), axis=1)
    ind_ref[...] = ind.astype(jnp.int32).reshape(ind_ref.shape)


def _dist_argmax(x_flat, embed2d):
    bn, d = x_flat.shape
    k = embed2d.shape[0]
    grid = (bn // _BLK,)
    dist, ind = pl.pallas_call(
        _dist_argmax_body,
        grid=grid,
        in_specs=[
            pl.BlockSpec((_BLK, d), lambda i: (i, 0)),
            pl.BlockSpec((k, d), lambda i: (0, 0)),
        ],
        out_specs=[
            pl.BlockSpec((_BLK, k), lambda i: (i, 0)),
            pl.BlockSpec((_BLK // 128, 128), lambda i: (i, 0)),
        ],
        out_shape=[
            jax.ShapeDtypeStruct((bn, k), jnp.float32),
            jax.ShapeDtypeStruct((bn // 128, 128), jnp.int32),
        ],
    )(x_flat, embed2d)
    return dist, ind.reshape(bn)


def _sc_gather(table, idx):
    info = plsc.get_sparse_core_info()
    nc, ns = info.num_cores, info.num_subcores
    nw = nc * ns
    bn = idx.shape[0]
    d = table.shape[1]
    b_per_w = bn // nw
    mesh = plsc.VectorSubcoreMesh(core_axis_name="c", subcore_axis_name="s")

    @functools.partial(
        pl.kernel, mesh=mesh,
        out_type=jax.ShapeDtypeStruct((bn, d), jnp.float32),
        scratch_types=[
            pltpu.VMEM((b_per_w,), jnp.int32),
            pltpu.VMEM((b_per_w, d), jnp.float32),
            pltpu.SemaphoreType.DMA,
        ],
        compiler_params=pltpu.CompilerParams(use_tc_tiling_on_sc=False),
    )
    def gk(table_hbm, idx_hbm, out_hbm, idx_v, rows_v, sem):
        wid = lax.axis_index("s") * nc + lax.axis_index("c")
        base = wid * b_per_w
        pltpu.sync_copy(idx_hbm.at[pl.ds(base, b_per_w)], idx_v)
        pltpu.async_copy(table_hbm.at[idx_v], rows_v, sem).wait()
        pltpu.sync_copy(rows_v, out_hbm.at[pl.ds(base, b_per_w)])

    return gk(table, idx)


def kernel(x, embed):
    b, n, d = x.shape
    h, k, _ = embed.shape
    x_flat = x.reshape(b * n, d).astype(jnp.float32)
    embed2d = embed.reshape(k, d)
    dist2d, ind = _dist_argmax(x_flat, embed2d)
    quantize = _sc_gather(embed2d, ind)
    return (quantize.reshape(b, n, d),
            ind.reshape(b, n),
            dist2d.reshape(h, b * n, k))
